# baseline (speedup 1.0000x reference)
import functools

import jax
import jax.numpy as jnp
from jax import lax
from jax.experimental import pallas as pl
from jax.experimental.pallas import tpu as pltpu
from jax.experimental.pallas import tpu_sc as plsc

M, D, H, B = 100000, 512, 2048, 16384

_NC, _NS = 2, 16
_NW = _NC * _NS
_BPW = B // _NW
_GCH = 64
_SCH = 64

_mesh = plsc.VectorSubcoreMesh(core_axis_name="c", subcore_axis_name="s")


_NGC = _BPW // _GCH


_NBUF = 3


@functools.partial(
    pl.kernel,
    out_type=jax.ShapeDtypeStruct((B, D), jnp.float32),
    mesh=_mesh,
    scratch_types=[
        pltpu.VMEM((_BPW,), jnp.int32),
        pltpu.VMEM((_NBUF, _GCH, D), jnp.float32),
    ]
    + [pltpu.SemaphoreType.DMA] * (2 * _NBUF),
)
def _sc_gather(mem_hbm, idx_hbm, out_hbm, idx_v, rows_v, *sems):
    wid = lax.axis_index("s") * _NC + lax.axis_index("c")
    base = wid * _BPW
    pltpu.sync_copy(idx_hbm.at[pl.ds(base, _BPW)], idx_v)
    sg, ss = sems[:_NBUF], sems[_NBUF:]

    def g_copy(c):
        b = c % _NBUF
        return pltpu.make_async_copy(
            mem_hbm.at[idx_v.at[pl.ds(c * _GCH, _GCH)]], rows_v.at[b], sg[b]
        )

    def s_copy(c):
        b = c % _NBUF
        return pltpu.make_async_copy(
            rows_v.at[b], out_hbm.at[pl.ds(base + c * _GCH, _GCH)], ss[b]
        )

    for c in range(min(_NBUF, _NGC)):
        g_copy(c).start()
    for c in range(_NGC):
        g_copy(c).wait()
        s_copy(c).start()
        if c + _NBUF < _NGC:
            s_copy(c).wait()
            g_copy(c + _NBUF).start()
    for c in range(max(0, _NGC - _NBUF), _NGC):
        s_copy(c).wait()


_BM = 1024
_GRID = B // _BM
_NCPY = 10
_CPR = M // _NCPY


def _mlp_body(mem_ref, x_ref, w1_ref, w2_ref, cpy_ref, res_ref, p_ref,
              acc_ref, csem):
    i = pl.program_id(0)

    @pl.when(i == 0)
    def _fire():
        for k in range(_NCPY):
            pltpu.make_async_copy(
                mem_ref.at[pl.ds(k * _CPR, _CPR)],
                cpy_ref.at[pl.ds(k * _CPR, _CPR)],
                csem,
            ).start()

    x = x_ref[...].astype(jnp.bfloat16)
    h = jnp.maximum(
        jnp.dot(x, w1_ref[...], preferred_element_type=jnp.float32),
        0.0,
    ).astype(jnp.bfloat16)
    r = jnp.dot(h, w2_ref[...], preferred_element_type=jnp.float32)
    res_ref[...] = r

    @pl.when(i == 0)
    def _init():
        acc_ref[0, 0] = 0.0

    acc_ref[0, 0] += jnp.sum(r)

    @pl.when(i == _GRID - 1)
    def _fin():
        p_ref[0, 0] = jax.nn.sigmoid(acc_ref[0, 0] / (B * D))
        for k in range(_NCPY):
            pltpu.make_async_copy(
                mem_ref.at[pl.ds(k * _CPR, _CPR)],
                cpy_ref.at[pl.ds(k * _CPR, _CPR)],
                csem,
            ).wait()


def _tc_mlp(mem, gathered, W1, W2):
    return pl.pallas_call(
        _mlp_body,
        grid=(_GRID,),
        in_specs=[
            pl.BlockSpec(memory_space=pltpu.MemorySpace.HBM),
            pl.BlockSpec((_BM, D), lambda i: (i, 0)),
            pl.BlockSpec((D, H), lambda i: (0, 0)),
            pl.BlockSpec((H, D), lambda i: (0, 0)),
        ],
        out_specs=[
            pl.BlockSpec(memory_space=pltpu.MemorySpace.HBM),
            pl.BlockSpec((_BM, D), lambda i: (i, 0)),
            pl.BlockSpec(memory_space=pltpu.SMEM),
        ],
        out_shape=[
            jax.ShapeDtypeStruct((M, D), jnp.float32),
            jax.ShapeDtypeStruct((B, D), jnp.float32),
            jax.ShapeDtypeStruct((1, 1), jnp.float32),
        ],
        scratch_shapes=[
            pltpu.SMEM((1, 1), jnp.float32),
            pltpu.SemaphoreType.DMA,
        ],
    )(mem, gathered, W1.astype(jnp.bfloat16), W2.astype(jnp.bfloat16))


_NCH = _BPW // _SCH
_MPAD = (M + 127) // 128 * 128


@functools.partial(
    pl.kernel,
    mesh=_mesh,
    scratch_types=[
        pltpu.VMEM((_NCH, _SCH), jnp.int32),
        pltpu.VMEM((_BPW,), jnp.int32),
        pltpu.VMEM((_NBUF, _SCH, D), jnp.float32),
    ]
    + [pltpu.SemaphoreType.DMA] * (2 * _NBUF),
)
def _sc_scatter(out_ref, res_hbm, tgt2d_hbm, src_hbm, tgt_v, src_v, rows_v, *sems):
    wid = lax.axis_index("s") * _NC + lax.axis_index("c")
    rbase = wid * _NCH
    pltpu.sync_copy(tgt2d_hbm.at[pl.ds(rbase, _NCH)], tgt_v)
    pltpu.sync_copy(src_hbm.at[pl.ds(wid * _BPW, _BPW)], src_v)
    sg, ss = sems[:_NBUF], sems[_NBUF:]

    def g_copy(c):
        b = c % _NBUF
        return pltpu.make_async_copy(
            res_hbm.at[src_v.at[pl.ds(c * _SCH, _SCH)]], rows_v.at[b], sg[b]
        )

    def s_copy(c):
        b = c % _NBUF
        return pltpu.make_async_copy(
            rows_v.at[b], out_ref.at[tgt_v.at[c]], ss[b]
        )

    for c in range(min(_NBUF, _NCH)):
        g_copy(c).start()
    for c in range(_NCH):
        g_copy(c).wait()
        s_copy(c).start()
        if c + _NBUF < _NCH:
            s_copy(c).wait()
            g_copy(c + _NBUF).start()
    for c in range(max(0, _NCH - _NBUF), _NCH):
        s_copy(c).wait()


@functools.partial(
    pl.kernel,
    out_type=jax.ShapeDtypeStruct((B,), jnp.int32),
    mesh=_mesh,
    scratch_types=[
        pltpu.VMEM((_MPAD,), jnp.int32),
        pltpu.VMEM((B,), jnp.int32),
        pltpu.VMEM((_BPW,), jnp.int32),
    ],
    compiler_params=pltpu.CompilerParams(needs_layout_passes=False),
)
def _sc_winner(tgt_hbm, src_hbm, win_v, tgt_v, src_v):
    wid = lax.axis_index("s") * _NC + lax.axis_index("c")
    pltpu.sync_copy(tgt_hbm, tgt_v)

    @pl.loop(0, B // 16)
    def _build(c):
        tv = tgt_v[pl.ds(c * 16, 16)]
        ids = c * 16 + lax.iota(jnp.int32, 16)
        _, is_last = plsc.scan_count(tv)
        plsc.store_scatter(win_v, [tv], ids, mask=is_last)

    base = wid * _BPW

    @pl.loop(0, _BPW // 16)
    def _emit(c):
        tv = tgt_v[pl.ds(base + c * 16, 16)]
        src_v[pl.ds(c * 16, 16)] = plsc.load_gather(win_v, [tv])

    pltpu.sync_copy(src_v, src_hbm.at[pl.ds(base, _BPW)])


def kernel(mem, W1, W2, prob, arg_idx, target_idx):
    gathered = _sc_gather(mem, arg_idx)
    src = _sc_winner(target_idx)
    new_mem0, res, p = _tc_mlp(mem, gathered, W1, W2)
    mem_ref = jax.new_ref(new_mem0)
    _sc_scatter(
        mem_ref,
        res,
        target_idx.reshape(_NW * _NCH, _SCH),
        src,
    )
    new_mem = mem_ref[...]
    new_prob = prob * p.reshape(1)
    return (new_mem, new_prob)


# speedup vs baseline: 27.2535x; 27.2535x over previous
import functools

import jax
import jax.numpy as jnp
from jax import lax
from jax.experimental import pallas as pl
from jax.experimental.pallas import tpu as pltpu
from jax.experimental.pallas import tpu_sc as plsc

M, D, H, B = 100000, 512, 2048, 16384

_NC, _NS = 2, 16
_NW = _NC * _NS
_BPW = B // _NW
_GCH = 64
_SCH = 64

_mesh = plsc.VectorSubcoreMesh(core_axis_name="c", subcore_axis_name="s")


_NGC = _BPW // _GCH


_NBUF = 3


@functools.partial(
    pl.kernel,
    out_type=jax.ShapeDtypeStruct((B, D), jnp.float32),
    mesh=_mesh,
    scratch_types=[
        pltpu.VMEM((_BPW,), jnp.int32),
        pltpu.VMEM((_NBUF, _GCH, D), jnp.float32),
    ]
    + [pltpu.SemaphoreType.DMA] * (2 * _NBUF),
)
def _sc_gather(mem_hbm, idx_hbm, out_hbm, idx_v, rows_v, *sems):
    wid = lax.axis_index("s") * _NC + lax.axis_index("c")
    base = wid * _BPW
    pltpu.sync_copy(idx_hbm.at[pl.ds(base, _BPW)], idx_v)
    sg, ss = sems[:_NBUF], sems[_NBUF:]

    def g_copy(c):
        b = c % _NBUF
        return pltpu.make_async_copy(
            mem_hbm.at[idx_v.at[pl.ds(c * _GCH, _GCH)]], rows_v.at[b], sg[b]
        )

    def s_copy(c):
        b = c % _NBUF
        return pltpu.make_async_copy(
            rows_v.at[b], out_hbm.at[pl.ds(base + c * _GCH, _GCH)], ss[b]
        )

    for c in range(min(_NBUF, _NGC)):
        g_copy(c).start()
    for c in range(_NGC):
        g_copy(c).wait()
        s_copy(c).start()
        if c + _NBUF < _NGC:
            s_copy(c).wait()
            g_copy(c + _NBUF).start()
    for c in range(max(0, _NGC - _NBUF), _NGC):
        s_copy(c).wait()


_BM = 512
_GRID = B // _BM
_CPB = 3200


def _mlp_body(mem_ref, x_ref, w1_ref, w2_ref, cpy_ref, res_ref, p_ref,
              acc_ref):
    i = pl.program_id(0)

    cpy_ref[...] = mem_ref[...]

    x = x_ref[...].astype(jnp.bfloat16)
    h = jnp.maximum(
        jnp.dot(x, w1_ref[...], preferred_element_type=jnp.float32),
        0.0,
    ).astype(jnp.bfloat16)
    r = jnp.dot(h, w2_ref[...], preferred_element_type=jnp.float32)
    res_ref[...] = r

    @pl.when(i == 0)
    def _init():
        acc_ref[0, 0] = 0.0

    acc_ref[0, 0] += jnp.sum(r)

    @pl.when(i == _GRID - 1)
    def _fin():
        p_ref[0, 0] = jax.nn.sigmoid(acc_ref[0, 0] / (B * D))


def _tc_mlp(mem, gathered, W1, W2):
    return pl.pallas_call(
        _mlp_body,
        grid=(_GRID,),
        in_specs=[
            pl.BlockSpec((_CPB, D), lambda i: (i, 0)),
            pl.BlockSpec((_BM, D), lambda i: (i, 0)),
            pl.BlockSpec((D, H), lambda i: (0, 0)),
            pl.BlockSpec((H, D), lambda i: (0, 0)),
        ],
        out_specs=[
            pl.BlockSpec((_CPB, D), lambda i: (i, 0)),
            pl.BlockSpec((_BM, D), lambda i: (i, 0)),
            pl.BlockSpec(memory_space=pltpu.SMEM),
        ],
        out_shape=[
            jax.ShapeDtypeStruct((M, D), jnp.float32),
            jax.ShapeDtypeStruct((B, D), jnp.float32),
            jax.ShapeDtypeStruct((1, 1), jnp.float32),
        ],
        scratch_shapes=[
            pltpu.SMEM((1, 1), jnp.float32),
        ],
    )(mem, gathered, W1.astype(jnp.bfloat16), W2.astype(jnp.bfloat16))


_NCH = _BPW // _SCH
_MPAD = (M + 127) // 128 * 128


@functools.partial(
    pl.kernel,
    mesh=_mesh,
    scratch_types=[
        pltpu.VMEM((_NCH, _SCH), jnp.int32),
        pltpu.VMEM((_BPW,), jnp.int32),
        pltpu.VMEM((_NBUF, _SCH, D), jnp.float32),
    ]
    + [pltpu.SemaphoreType.DMA] * (2 * _NBUF),
)
def _sc_scatter(out_ref, res_hbm, tgt2d_hbm, src_hbm, tgt_v, src_v, rows_v, *sems):
    wid = lax.axis_index("s") * _NC + lax.axis_index("c")
    rbase = wid * _NCH
    pltpu.sync_copy(tgt2d_hbm.at[pl.ds(rbase, _NCH)], tgt_v)
    pltpu.sync_copy(src_hbm.at[pl.ds(wid * _BPW, _BPW)], src_v)
    sg, ss = sems[:_NBUF], sems[_NBUF:]

    def g_copy(c):
        b = c % _NBUF
        return pltpu.make_async_copy(
            res_hbm.at[src_v.at[pl.ds(c * _SCH, _SCH)]], rows_v.at[b], sg[b]
        )

    def s_copy(c):
        b = c % _NBUF
        return pltpu.make_async_copy(
            rows_v.at[b], out_ref.at[tgt_v.at[c]], ss[b]
        )

    for c in range(min(_NBUF, _NCH)):
        g_copy(c).start()
    for c in range(_NCH):
        g_copy(c).wait()
        s_copy(c).start()
        if c + _NBUF < _NCH:
            s_copy(c).wait()
            g_copy(c + _NBUF).start()
    for c in range(max(0, _NCH - _NBUF), _NCH):
        s_copy(c).wait()


@functools.partial(
    pl.kernel,
    out_type=jax.ShapeDtypeStruct((B,), jnp.int32),
    mesh=_mesh,
    scratch_types=[
        pltpu.VMEM((_MPAD,), jnp.int32),
        pltpu.VMEM((B,), jnp.int32),
        pltpu.VMEM((_BPW,), jnp.int32),
    ],
    compiler_params=pltpu.CompilerParams(needs_layout_passes=False),
)
def _sc_winner(tgt_hbm, src_hbm, win_v, tgt_v, src_v):
    wid = lax.axis_index("s") * _NC + lax.axis_index("c")
    pltpu.sync_copy(tgt_hbm, tgt_v)

    @pl.loop(0, B // 16)
    def _build(c):
        tv = tgt_v[pl.ds(c * 16, 16)]
        ids = c * 16 + lax.iota(jnp.int32, 16)
        _, is_last = plsc.scan_count(tv)
        plsc.store_scatter(win_v, [tv], ids, mask=is_last)

    base = wid * _BPW

    @pl.loop(0, _BPW // 16)
    def _emit(c):
        tv = tgt_v[pl.ds(base + c * 16, 16)]
        src_v[pl.ds(c * 16, 16)] = plsc.load_gather(win_v, [tv])

    pltpu.sync_copy(src_v, src_hbm.at[pl.ds(base, _BPW)])


def kernel(mem, W1, W2, prob, arg_idx, target_idx):
    gathered = _sc_gather(mem, arg_idx)
    src = _sc_winner(target_idx)
    new_mem0, res, p = _tc_mlp(mem, gathered, W1, W2)
    mem_ref = jax.new_ref(new_mem0)
    _sc_scatter(
        mem_ref,
        res,
        target_idx.reshape(_NW * _NCH, _SCH),
        src,
    )
    new_mem = mem_ref[...]
    new_prob = prob * p.reshape(1)
    return (new_mem, new_prob)


# speedup vs baseline: 27.3043x; 1.0019x over previous
import functools

import jax
import jax.numpy as jnp
from jax import lax
from jax.experimental import pallas as pl
from jax.experimental.pallas import tpu as pltpu
from jax.experimental.pallas import tpu_sc as plsc

M, D, H, B = 100000, 512, 2048, 16384

_NC, _NS = 2, 16
_NW = _NC * _NS
_BPW = B // _NW
_GCH = 32
_SCH = 64

_mesh = plsc.VectorSubcoreMesh(core_axis_name="c", subcore_axis_name="s")


_NGC = _BPW // _GCH


_NBUF = 4


@functools.partial(
    pl.kernel,
    out_type=jax.ShapeDtypeStruct((B, D), jnp.float32),
    mesh=_mesh,
    scratch_types=[
        pltpu.VMEM((_BPW,), jnp.int32),
        pltpu.VMEM((_NBUF, _GCH, D), jnp.float32),
    ]
    + [pltpu.SemaphoreType.DMA] * (2 * _NBUF),
)
def _sc_gather(mem_hbm, idx_hbm, out_hbm, idx_v, rows_v, *sems):
    wid = lax.axis_index("s") * _NC + lax.axis_index("c")
    base = wid * _BPW
    pltpu.sync_copy(idx_hbm.at[pl.ds(base, _BPW)], idx_v)
    sg, ss = sems[:_NBUF], sems[_NBUF:]

    def g_copy(c):
        b = c % _NBUF
        return pltpu.make_async_copy(
            mem_hbm.at[idx_v.at[pl.ds(c * _GCH, _GCH)]], rows_v.at[b], sg[b]
        )

    def s_copy(c):
        b = c % _NBUF
        return pltpu.make_async_copy(
            rows_v.at[b], out_hbm.at[pl.ds(base + c * _GCH, _GCH)], ss[b]
        )

    for c in range(min(_NBUF, _NGC)):
        g_copy(c).start()
    for c in range(_NGC):
        g_copy(c).wait()
        s_copy(c).start()
        if c + _NBUF < _NGC:
            s_copy(c).wait()
            g_copy(c + _NBUF).start()
    for c in range(max(0, _NGC - _NBUF), _NGC):
        s_copy(c).wait()


_BM = 512
_GRID = B // _BM
_CPB = 3200


def _mlp_body(mem_ref, x_ref, w1_ref, w2_ref, cpy_ref, res_ref, p_ref,
              acc_ref):
    i = pl.program_id(0)

    cpy_ref[...] = mem_ref[...]

    x = x_ref[...].astype(jnp.bfloat16)
    h = jnp.maximum(
        jnp.dot(x, w1_ref[...], preferred_element_type=jnp.float32),
        0.0,
    ).astype(jnp.bfloat16)
    r = jnp.dot(h, w2_ref[...], preferred_element_type=jnp.float32)
    res_ref[...] = r

    @pl.when(i == 0)
    def _init():
        acc_ref[0, 0] = 0.0

    acc_ref[0, 0] += jnp.sum(r)

    @pl.when(i == _GRID - 1)
    def _fin():
        p_ref[0, 0] = jax.nn.sigmoid(acc_ref[0, 0] / (B * D))


def _tc_mlp(mem, gathered, W1, W2):
    return pl.pallas_call(
        _mlp_body,
        grid=(_GRID,),
        in_specs=[
            pl.BlockSpec((_CPB, D), lambda i: (i, 0)),
            pl.BlockSpec((_BM, D), lambda i: (i, 0)),
            pl.BlockSpec((D, H), lambda i: (0, 0)),
            pl.BlockSpec((H, D), lambda i: (0, 0)),
        ],
        out_specs=[
            pl.BlockSpec((_CPB, D), lambda i: (i, 0)),
            pl.BlockSpec((_BM, D), lambda i: (i, 0)),
            pl.BlockSpec(memory_space=pltpu.SMEM),
        ],
        out_shape=[
            jax.ShapeDtypeStruct((M, D), jnp.float32),
            jax.ShapeDtypeStruct((B, D), jnp.float32),
            jax.ShapeDtypeStruct((1, 1), jnp.float32),
        ],
        scratch_shapes=[
            pltpu.SMEM((1, 1), jnp.float32),
        ],
    )(mem, gathered, W1.astype(jnp.bfloat16), W2.astype(jnp.bfloat16))


_NCH = _BPW // _SCH
_SNBUF = 3
_MPAD = (M + 127) // 128 * 128


@functools.partial(
    pl.kernel,
    mesh=_mesh,
    scratch_types=[
        pltpu.VMEM((_NCH, _SCH), jnp.int32),
        pltpu.VMEM((_BPW,), jnp.int32),
        pltpu.VMEM((_SNBUF, _SCH, D), jnp.float32),
    ]
    + [pltpu.SemaphoreType.DMA] * (2 * _SNBUF),
)
def _sc_scatter(out_ref, res_hbm, tgt2d_hbm, src_hbm, tgt_v, src_v, rows_v, *sems):
    wid = lax.axis_index("s") * _NC + lax.axis_index("c")
    rbase = wid * _NCH
    pltpu.sync_copy(tgt2d_hbm.at[pl.ds(rbase, _NCH)], tgt_v)
    pltpu.sync_copy(src_hbm.at[pl.ds(wid * _BPW, _BPW)], src_v)
    sg, ss = sems[:_SNBUF], sems[_SNBUF:]

    def g_copy(c):
        b = c % _SNBUF
        return pltpu.make_async_copy(
            res_hbm.at[src_v.at[pl.ds(c * _SCH, _SCH)]], rows_v.at[b], sg[b]
        )

    def s_copy(c):
        b = c % _SNBUF
        return pltpu.make_async_copy(
            rows_v.at[b], out_ref.at[tgt_v.at[c]], ss[b]
        )

    for c in range(min(_SNBUF, _NCH)):
        g_copy(c).start()
    for c in range(_NCH):
        g_copy(c).wait()
        s_copy(c).start()
        if c + _SNBUF < _NCH:
            s_copy(c).wait()
            g_copy(c + _SNBUF).start()
    for c in range(max(0, _NCH - _SNBUF), _NCH):
        s_copy(c).wait()


@functools.partial(
    pl.kernel,
    out_type=jax.ShapeDtypeStruct((B,), jnp.int32),
    mesh=_mesh,
    scratch_types=[
        pltpu.VMEM((_MPAD,), jnp.int32),
        pltpu.VMEM((B,), jnp.int32),
        pltpu.VMEM((_BPW,), jnp.int32),
    ],
    compiler_params=pltpu.CompilerParams(needs_layout_passes=False),
)
def _sc_winner(tgt_hbm, src_hbm, win_v, tgt_v, src_v):
    wid = lax.axis_index("s") * _NC + lax.axis_index("c")
    pltpu.sync_copy(tgt_hbm, tgt_v)

    @pl.loop(0, B // 16)
    def _build(c):
        tv = tgt_v[pl.ds(c * 16, 16)]
        ids = c * 16 + lax.iota(jnp.int32, 16)
        _, is_last = plsc.scan_count(tv)
        plsc.store_scatter(win_v, [tv], ids, mask=is_last)

    base = wid * _BPW

    @pl.loop(0, _BPW // 16)
    def _emit(c):
        tv = tgt_v[pl.ds(base + c * 16, 16)]
        src_v[pl.ds(c * 16, 16)] = plsc.load_gather(win_v, [tv])

    pltpu.sync_copy(src_v, src_hbm.at[pl.ds(base, _BPW)])


def kernel(mem, W1, W2, prob, arg_idx, target_idx):
    gathered = _sc_gather(mem, arg_idx)
    src = _sc_winner(target_idx)
    new_mem0, res, p = _tc_mlp(mem, gathered, W1, W2)
    mem_ref = jax.new_ref(new_mem0)
    _sc_scatter(
        mem_ref,
        res,
        target_idx.reshape(_NW * _NCH, _SCH),
        src,
    )
    new_mem = mem_ref[...]
    new_prob = prob * p.reshape(1)
    return (new_mem, new_prob)
